# baseline (speedup 1.0000x reference)
import jax
import jax.numpy as jnp
from jax.experimental import pallas as pl
from jax.experimental.pallas import tpu as pltpu


def _copy3_kernel(c_in, n_in, u_in, c_out, n_out, u_out, c_sem, n_sem, u_sem):
    c = pltpu.make_async_copy(c_in, c_out, c_sem)
    n = pltpu.make_async_copy(n_in, n_out, n_sem)
    u = pltpu.make_async_copy(u_in, u_out, u_sem)
    c.start()
    n.start()
    u.start()
    c.wait()
    n.wait()
    u.wait()


def kernel(c_embeddings, n_embeddings, u_embeddings):
    out = pl.pallas_call(
        _copy3_kernel,
        in_specs=[pl.BlockSpec(memory_space=pl.ANY)] * 3,
        out_specs=[pl.BlockSpec(memory_space=pl.ANY)] * 3,
        out_shape=(
            jax.ShapeDtypeStruct(c_embeddings.shape, c_embeddings.dtype),
            jax.ShapeDtypeStruct(n_embeddings.shape, n_embeddings.dtype),
            jax.ShapeDtypeStruct(u_embeddings.shape, u_embeddings.dtype),
        ),
        scratch_shapes=[pltpu.SemaphoreType.DMA] * 3,
    )(c_embeddings, n_embeddings, u_embeddings)
    return (out[0], out[1], out[2])


# speedup vs baseline: 28.1949x; 28.1949x over previous
import jax
import jax.numpy as jnp
from jax.experimental import pallas as pl
from jax.experimental.pallas import tpu as pltpu


_ROWS = 100000
_BLK = 4000


def _copy3_kernel(c_in, n_in, u_in, c_out, n_out, u_out):
    c_out[...] = c_in[...]
    n_out[...] = n_in[...]
    u_out[...] = u_in[...]


def kernel(c_embeddings, n_embeddings, u_embeddings):
    grid = (_ROWS // _BLK,)
    out = pl.pallas_call(
        _copy3_kernel,
        grid=grid,
        in_specs=[
            pl.BlockSpec((_BLK, 128), lambda i: (i, 0)),
            pl.BlockSpec((_BLK, 128), lambda i: (i, 0)),
            pl.BlockSpec((_BLK, 64), lambda i: (i, 0)),
        ],
        out_specs=[
            pl.BlockSpec((_BLK, 128), lambda i: (i, 0)),
            pl.BlockSpec((_BLK, 128), lambda i: (i, 0)),
            pl.BlockSpec((_BLK, 64), lambda i: (i, 0)),
        ],
        out_shape=(
            jax.ShapeDtypeStruct(c_embeddings.shape, c_embeddings.dtype),
            jax.ShapeDtypeStruct(n_embeddings.shape, n_embeddings.dtype),
            jax.ShapeDtypeStruct(u_embeddings.shape, u_embeddings.dtype),
        ),
    )(c_embeddings, n_embeddings, u_embeddings)
    return (out[0], out[1], out[2])
